# baseline (speedup 1.0000x reference)
import functools

import jax
import jax.numpy as jnp
from jax import lax
from jax.experimental import pallas as pl
from jax.experimental.pallas import tpu as pltpu
from jax.experimental.pallas import tpu_sc as plsc

V = 100000
D = 128
B = 4096 * 200
NC = 2
NS = 16
NW = NC * NS
BPW = B // NW
CH = 64
NBUF = 4
NCH = BPW // CH

assert NCH % NBUF == 0

_mesh = plsc.VectorSubcoreMesh(core_axis_name="c", subcore_axis_name="s")


@functools.partial(
    pl.kernel,
    mesh=_mesh,
    out_type=jax.ShapeDtypeStruct((B, D), jnp.float32),
    scratch_types=(
        [pltpu.VMEM((BPW,), jnp.int32)]
        + [pltpu.VMEM((CH, D), jnp.float32) for _ in range(NBUF)]
        + [pltpu.VMEM_SHARED((NS, NBUF, CH, D), jnp.float32)]
        + [pltpu.SemaphoreType.DMA for _ in range(3 * NBUF)]
    ),
)
def _sc_gather(tok_hbm, codes_hbm, out_hbm, idx_v, *rest):
    bufs = rest[:NBUF]
    spm = rest[NBUF]
    sem_g = rest[NBUF + 1 : 2 * NBUF + 1]
    sem_x = rest[2 * NBUF + 1 : 3 * NBUF + 1]
    sem_w = rest[3 * NBUF + 1 :]
    sid = lax.axis_index("s")
    wid = sid * NC + lax.axis_index("c")
    base = wid * BPW

    pltpu.sync_copy(tok_hbm.at[pl.ds(base, BPW)], idx_v)

    def start_gather(c, b):
        pltpu.async_copy(
            codes_hbm.at[idx_v.at[pl.ds(c * CH, CH)]], bufs[b], sem_g[b]
        )

    def wait_gather(c, b):
        pltpu.make_async_copy(
            codes_hbm.at[idx_v.at[pl.ds(c * CH, CH)]], bufs[b], sem_g[b]
        ).wait()

    def start_xbar(b):
        pltpu.async_copy(bufs[b], spm.at[sid, b], sem_x[b])

    def wait_xbar(b):
        pltpu.make_async_copy(bufs[b], spm.at[sid, b], sem_x[b]).wait()

    def start_write(c, b):
        pltpu.async_copy(
            spm.at[sid, b], out_hbm.at[pl.ds(base + c * CH, CH)], sem_w[b]
        )

    def wait_write(c, b):
        pltpu.make_async_copy(
            spm.at[sid, b], out_hbm.at[pl.ds(base + c * CH, CH)], sem_w[b]
        ).wait()

    for b in range(2):
        start_gather(b, b)

    def group(gi, carry):
        c0 = gi * NBUF
        for b in range(NBUF):
            c = c0 + b
            pb = (b + NBUF - 1) % NBUF

            wait_gather(c, b)

            @pl.when(c >= NBUF)
            def _():
                wait_write(c - NBUF, b)

            start_xbar(b)

            @pl.when(c >= 1)
            def _():
                wait_xbar(pb)
                start_write(c - 1, pb)

            @pl.when(c + 2 < NCH)
            def _():
                start_gather(c + 2, (b + 2) % NBUF)

        return carry

    lax.fori_loop(0, NCH // NBUF, group, 0)

    lastb = (NCH - 1) % NBUF
    wait_xbar(lastb)
    start_write(NCH - 1, lastb)
    for c in range(NCH - NBUF, NCH):
        wait_write(c, c % NBUF)


def kernel(tokens, codes):
    idx = tokens.reshape(-1).astype(jnp.int32)
    out = _sc_gather(idx, codes)
    return out.reshape(tokens.shape + (D,))


# speedup vs baseline: 1.0154x; 1.0154x over previous
import functools

import jax
import jax.numpy as jnp
from jax import lax
from jax.experimental import pallas as pl
from jax.experimental.pallas import tpu as pltpu
from jax.experimental.pallas import tpu_sc as plsc

V = 100000
D = 128
B = 4096 * 200
NC = 2
NS = 16
NW = NC * NS
BPW = B // NW
CH = 128
NBUF = 4
NCH = BPW // CH

assert NCH % NBUF == 0

_mesh = plsc.VectorSubcoreMesh(core_axis_name="c", subcore_axis_name="s")


@functools.partial(
    pl.kernel,
    mesh=_mesh,
    out_type=jax.ShapeDtypeStruct((B, D), jnp.float32),
    scratch_types=(
        [pltpu.VMEM((BPW,), jnp.int32)]
        + [pltpu.VMEM((CH, D), jnp.float32) for _ in range(NBUF)]
        + [pltpu.SemaphoreType.DMA for _ in range(NBUF)]
    ),
)
def _sc_gather(tok_hbm, codes_hbm, out_hbm, idx_v, *bufs_sems):
    bufs = bufs_sems[:NBUF]
    sems = bufs_sems[NBUF:]
    wid = lax.axis_index("s") * NC + lax.axis_index("c")
    base = wid * BPW

    pltpu.sync_copy(tok_hbm.at[pl.ds(base, BPW)], idx_v)

    for b in range(NBUF):
        pltpu.async_copy(
            codes_hbm.at[idx_v.at[pl.ds(b * CH, CH)]], bufs[b], sems[b]
        )

    def group(gi, carry):
        c0 = gi * NBUF
        for b in range(NBUF):
            c = c0 + b
            pltpu.make_async_copy(
                codes_hbm.at[idx_v.at[pl.ds(c * CH, CH)]], bufs[b], sems[b]
            ).wait()
            pltpu.sync_copy(bufs[b], out_hbm.at[pl.ds(base + c * CH, CH)])
            nxt = c + NBUF

            @pl.when(nxt < NCH)
            def _():
                pltpu.async_copy(
                    codes_hbm.at[idx_v.at[pl.ds(nxt * CH, CH)]], bufs[b], sems[b]
                )

        return carry

    lax.fori_loop(0, NCH // NBUF, group, 0)


def kernel(tokens, codes):
    idx = tokens.reshape(-1).astype(jnp.int32)
    out = _sc_gather(idx, codes)
    return out.reshape(tokens.shape + (D,))
